# baseline (speedup 1.0000x reference)
import functools

import jax
import jax.numpy as jnp
from jax import lax
from jax.experimental import pallas as pl
from jax.experimental.pallas import tpu as pltpu
from jax.experimental.pallas import tpu_sc as plsc

D = 768
E = 8
NT = 2048
BT = 256
NTILES = NT // BT
W_MAX = NTILES + E - 1
LSUB = 128
NC, NS = 2, 16
NW = NC * NS
CHUNK = NT // NW

_INV_SQRT2 = 0.7071067811865476


def _gelu_exact(h):
    return 0.5 * h * (1.0 + lax.erf(h * _INV_SQRT2))


WPAD = 32


def _gating_body(x_ref, gw_ref, gb_ref, pos_ref, wrow_ref, meta_ref):
    x = x_ref[...]
    logits = jnp.dot(x, gw_ref[...], preferred_element_type=jnp.float32)
    logits = logits + gb_ref[...]
    m = jnp.max(logits, axis=-1, keepdims=True)
    ex = jnp.exp(logits - m)
    probs = ex / jnp.sum(ex, axis=-1, keepdims=True)
    assign = jnp.argmax(probs, axis=-1)
    wmax = jnp.max(probs, axis=-1, keepdims=True)
    wrow_ref[...] = jnp.broadcast_to(wmax, (NT, LSUB))

    onehot = (lax.broadcasted_iota(jnp.int32, (NT, E), 1)
              == assign[:, None]).astype(jnp.int32)
    c = onehot
    k = 1
    while k < NT:
        shifted = jnp.concatenate(
            [jnp.zeros((k, E), jnp.int32), c[: NT - k, :]], axis=0)
        c = c + shifted
        k *= 2
    counts_row = c[NT - 1:NT, :]
    def _shift_lanes(v, k):
        return jnp.concatenate(
            [jnp.zeros((1, k), jnp.int32), v[:, : E - k]], axis=1)

    offs = _shift_lanes(counts_row, 1)
    offs = offs + _shift_lanes(offs, 1)
    offs = offs + _shift_lanes(offs, 2)
    offs = offs + _shift_lanes(offs, 4)
    pos = jnp.sum(onehot * (c - 1 + offs), axis=1, keepdims=True)
    pos_ref[...] = pos

    incl = offs + counts_row
    ts = offs // BT
    te = jnp.where(counts_row > 0, (incl + BT - 1) // BT, ts)
    ntl = te - ts
    incl_t = ntl + _shift_lanes(ntl, 1)
    incl_t = incl_t + _shift_lanes(incl_t, 2)
    incl_t = incl_t + _shift_lanes(incl_t, 4)
    excl_t = incl_t - ntl

    eye = (lax.broadcasted_iota(jnp.int32, (E, E), 0)
           == lax.broadcasted_iota(jnp.int32, (E, E), 1)).astype(jnp.int32)

    def to_sub(v):
        return jnp.sum(jnp.broadcast_to(v, (E, E)) * eye, axis=1,
                       keepdims=True)

    wl = lax.broadcasted_iota(jnp.int32, (E, WPAD), 1)
    sub = lax.broadcasted_iota(jnp.int32, (E, WPAD), 0)
    g = jnp.minimum(
        jnp.sum((wl >= to_sub(incl_t)).astype(jnp.int32), axis=0,
                keepdims=True), E - 1)
    sel = (sub == jnp.broadcast_to(g, (E, WPAD))).astype(jnp.int32)

    def gath(v):
        return jnp.sum(to_sub(v) * sel, axis=0, keepdims=True)

    wr = lax.broadcasted_iota(jnp.int32, (1, WPAD), 1)
    tt = gath(ts) + (wr - gath(excl_t))
    valid = wr < jnp.sum(ntl, axis=1, keepdims=True)
    tt = jnp.where(valid, tt, NTILES - 1)
    rs = jnp.where(valid, jnp.maximum(gath(offs), tt * BT) - tt * BT, 0)
    re = jnp.where(valid,
                   jnp.minimum(gath(incl), tt * BT + BT) - tt * BT, 0)
    meta_ref[...] = jnp.concatenate([g, tt, rs, re], axis=0)


def _gating(x_flat, gate_w, gate_b):
    return pl.pallas_call(
        _gating_body,
        in_specs=[
            pl.BlockSpec((NT, D), lambda: (0, 0)),
            pl.BlockSpec((D, E), lambda: (0, 0)),
            pl.BlockSpec((1, E), lambda: (0, 0)),
        ],
        out_specs=[
            pl.BlockSpec((NT, 1), lambda: (0, 0)),
            pl.BlockSpec((NT, LSUB), lambda: (0, 0)),
            pl.BlockSpec((4, WPAD), lambda: (0, 0)),
        ],
        out_shape=[
            jax.ShapeDtypeStruct((NT, 1), jnp.int32),
            jax.ShapeDtypeStruct((NT, LSUB), jnp.float32),
            jax.ShapeDtypeStruct((4, WPAD), jnp.int32),
        ],
    )(x_flat, gate_w, gate_b.reshape(1, E))


def _sc_worker_base():
    wid = lax.axis_index("s") * NC + lax.axis_index("c")
    return wid * CHUNK


@functools.cache
def _sc_kernels():
    mesh = plsc.VectorSubcoreMesh(
        core_axis_name="c", subcore_axis_name="s",
        num_cores=NC, num_subcores=NS)

    @functools.partial(
        pl.kernel,
        out_type=[
            jax.ShapeDtypeStruct((NT, D), jnp.float32),
            jax.ShapeDtypeStruct((NT, LSUB), jnp.float32),
        ],
        mesh=mesh,
        scratch_types=[
            pltpu.VMEM((CHUNK,), jnp.int32),
            pltpu.VMEM((CHUNK, D), jnp.float32),
            pltpu.VMEM((CHUNK, LSUB), jnp.float32),
            pltpu.SemaphoreType.DMA,
            pltpu.SemaphoreType.DMA,
        ],
    )
    def sc_scatter(x_hbm, pos_hbm, wrow_hbm, xs_hbm, ws_hbm,
                   idx_v, rows_v, wv, sem1, sem2):
        base = _sc_worker_base()
        pltpu.sync_copy(pos_hbm.at[pl.ds(base, CHUNK)], idx_v)
        pltpu.sync_copy(x_hbm.at[pl.ds(base, CHUNK)], rows_v)
        pltpu.sync_copy(wrow_hbm.at[pl.ds(base, CHUNK)], wv)
        cp1 = pltpu.async_copy(rows_v, xs_hbm.at[idx_v], sem1)
        cp2 = pltpu.async_copy(wv, ws_hbm.at[idx_v], sem2)
        cp1.wait()
        cp2.wait()

    @functools.partial(
        pl.kernel,
        out_type=jax.ShapeDtypeStruct((NT, D), jnp.float32),
        mesh=mesh,
        scratch_types=[
            pltpu.VMEM((CHUNK,), jnp.int32),
            pltpu.VMEM((CHUNK, D), jnp.float32),
            pltpu.SemaphoreType.DMA,
        ],
    )
    def sc_gather(y_hbm, pos_hbm, out_hbm, idx_v, rows_v, sem):
        base = _sc_worker_base()
        pltpu.sync_copy(pos_hbm.at[pl.ds(base, CHUNK)], idx_v)
        pltpu.async_copy(y_hbm.at[idx_v], rows_v, sem).wait()
        pltpu.sync_copy(rows_v, out_hbm.at[pl.ds(base, CHUNK)])

    return sc_scatter, sc_gather


def _ffn_body(meta_ref,
              xs_ref, w1_ref, b1_ref, w2_ref, b2_ref, ws_ref, out_ref):
    w = pl.program_id(0)
    rs = meta_ref[2, w]
    re = meta_ref[3, w]

    @pl.when(rs < re)
    def _work():
        xt = xs_ref[...].astype(jnp.bfloat16)
        h = jnp.dot(xt, w1_ref[0].astype(jnp.bfloat16),
                    preferred_element_type=jnp.float32)
        h = _gelu_exact(h + b1_ref[0])
        y = jnp.dot(h.astype(jnp.bfloat16), w2_ref[0].astype(jnp.bfloat16),
                    preferred_element_type=jnp.float32)
        y = (y + b2_ref[0]) * ws_ref[...][:, :1]
        row = lax.broadcasted_iota(jnp.int32, (BT, 1), 0)
        mask = (row >= rs) & (row < re)
        out_ref[...] = jnp.where(mask, y, out_ref[...])


def _grouped_ffn(meta, x_sorted, w1, b1, w2, b2, w_sorted):
    grid_spec = pltpu.PrefetchScalarGridSpec(
        num_scalar_prefetch=1,
        grid=(W_MAX,),
        in_specs=[
            pl.BlockSpec((BT, D), lambda w, m: (m[1, w], 0)),
            pl.BlockSpec((1, D, D), lambda w, m: (m[0, w], 0, 0)),
            pl.BlockSpec((1, 1, D), lambda w, m: (m[0, w], 0, 0)),
            pl.BlockSpec((1, D, D), lambda w, m: (m[0, w], 0, 0)),
            pl.BlockSpec((1, 1, D), lambda w, m: (m[0, w], 0, 0)),
            pl.BlockSpec((BT, LSUB), lambda w, m: (m[1, w], 0)),
        ],
        out_specs=pl.BlockSpec((BT, D), lambda w, m: (m[1, w], 0)),
    )
    return pl.pallas_call(
        _ffn_body,
        grid_spec=grid_spec,
        out_shape=jax.ShapeDtypeStruct((NT, D), jnp.float32),
        compiler_params=pltpu.CompilerParams(
            dimension_semantics=("arbitrary",),
        ),
    )(meta, x_sorted, w1, b1.reshape(E, 1, D), w2, b2.reshape(E, 1, D),
      w_sorted)


def kernel(x, gate_w, gate_b, w1, b1, w2, b2):
    n, t, d = x.shape
    x_flat = x.reshape(n * t, d)

    sc_scatter, sc_gather = _sc_kernels()
    pos2d, wrow, meta = _gating(x_flat, gate_w, gate_b)
    pos = pos2d.reshape(NT)
    x_sorted, w_sorted = sc_scatter(x_flat, pos, wrow)
    y_sorted = _grouped_ffn(meta, x_sorted, w1, b1, w2, b2, w_sorted)
    out = sc_gather(y_sorted, pos)
    return out.reshape(n, t, d)


# speedup vs baseline: 1.0422x; 1.0422x over previous
import functools

import jax
import jax.numpy as jnp
from jax import lax
from jax.experimental import pallas as pl
from jax.experimental.pallas import tpu as pltpu
from jax.experimental.pallas import tpu_sc as plsc

D = 768
E = 8
NT = 2048
BT = 512
NTILES = NT // BT
W_MAX = NTILES + E - 1
LSUB = 128
NC, NS = 2, 16
NW = NC * NS
CHUNK = NT // NW

_INV_SQRT2 = 0.7071067811865476


def _gelu_exact(h):
    return 0.5 * h * (1.0 + lax.erf(h * _INV_SQRT2))


WPAD = 32


def _gating_body(x_ref, gw_ref, gb_ref, pos_ref, wrow_ref, meta_ref):
    x = x_ref[...]
    logits = jnp.dot(x, gw_ref[...], preferred_element_type=jnp.float32)
    logits = logits + gb_ref[...]
    m = jnp.max(logits, axis=-1, keepdims=True)
    ex = jnp.exp(logits - m)
    probs = ex / jnp.sum(ex, axis=-1, keepdims=True)
    assign = jnp.argmax(probs, axis=-1)
    wmax = jnp.max(probs, axis=-1, keepdims=True)
    wrow_ref[...] = jnp.broadcast_to(wmax, (NT, LSUB))

    onehot = (lax.broadcasted_iota(jnp.int32, (NT, E), 1)
              == assign[:, None]).astype(jnp.int32)
    c = onehot
    k = 1
    while k < NT:
        shifted = jnp.concatenate(
            [jnp.zeros((k, E), jnp.int32), c[: NT - k, :]], axis=0)
        c = c + shifted
        k *= 2
    counts_row = c[NT - 1:NT, :]
    def _shift_lanes(v, k):
        return jnp.concatenate(
            [jnp.zeros((1, k), jnp.int32), v[:, : E - k]], axis=1)

    offs = _shift_lanes(counts_row, 1)
    offs = offs + _shift_lanes(offs, 1)
    offs = offs + _shift_lanes(offs, 2)
    offs = offs + _shift_lanes(offs, 4)
    pos = jnp.sum(onehot * (c - 1 + offs), axis=1, keepdims=True)
    pos_ref[...] = pos

    incl = offs + counts_row
    ts = offs // BT
    te = jnp.where(counts_row > 0, (incl + BT - 1) // BT, ts)
    ntl = te - ts
    incl_t = ntl + _shift_lanes(ntl, 1)
    incl_t = incl_t + _shift_lanes(incl_t, 2)
    incl_t = incl_t + _shift_lanes(incl_t, 4)
    excl_t = incl_t - ntl

    eye = (lax.broadcasted_iota(jnp.int32, (E, E), 0)
           == lax.broadcasted_iota(jnp.int32, (E, E), 1)).astype(jnp.int32)

    def to_sub(v):
        return jnp.sum(jnp.broadcast_to(v, (E, E)) * eye, axis=1,
                       keepdims=True)

    wl = lax.broadcasted_iota(jnp.int32, (E, WPAD), 1)
    sub = lax.broadcasted_iota(jnp.int32, (E, WPAD), 0)
    g = jnp.minimum(
        jnp.sum((wl >= to_sub(incl_t)).astype(jnp.int32), axis=0,
                keepdims=True), E - 1)
    sel = (sub == jnp.broadcast_to(g, (E, WPAD))).astype(jnp.int32)

    def gath(v):
        return jnp.sum(to_sub(v) * sel, axis=0, keepdims=True)

    wr = lax.broadcasted_iota(jnp.int32, (1, WPAD), 1)
    tt = gath(ts) + (wr - gath(excl_t))
    valid = wr < jnp.sum(ntl, axis=1, keepdims=True)
    tt = jnp.where(valid, tt, NTILES - 1)
    rs = jnp.where(valid, jnp.maximum(gath(offs), tt * BT) - tt * BT, 0)
    re = jnp.where(valid,
                   jnp.minimum(gath(incl), tt * BT + BT) - tt * BT, 0)
    meta_ref[...] = jnp.concatenate([g, tt, rs, re], axis=0)


def _gating(x_flat, gate_w, gate_b):
    return pl.pallas_call(
        _gating_body,
        in_specs=[
            pl.BlockSpec((NT, D), lambda: (0, 0)),
            pl.BlockSpec((D, E), lambda: (0, 0)),
            pl.BlockSpec((1, E), lambda: (0, 0)),
        ],
        out_specs=[
            pl.BlockSpec((NT, 1), lambda: (0, 0)),
            pl.BlockSpec((NT, LSUB), lambda: (0, 0)),
            pl.BlockSpec((4, WPAD), lambda: (0, 0)),
        ],
        out_shape=[
            jax.ShapeDtypeStruct((NT, 1), jnp.int32),
            jax.ShapeDtypeStruct((NT, LSUB), jnp.float32),
            jax.ShapeDtypeStruct((4, WPAD), jnp.int32),
        ],
    )(x_flat, gate_w, gate_b.reshape(1, E))


def _sc_worker_base():
    wid = lax.axis_index("s") * NC + lax.axis_index("c")
    return wid * CHUNK


@functools.cache
def _sc_kernels():
    mesh = plsc.VectorSubcoreMesh(
        core_axis_name="c", subcore_axis_name="s",
        num_cores=NC, num_subcores=NS)

    @functools.partial(
        pl.kernel,
        out_type=[
            jax.ShapeDtypeStruct((NT, D), jnp.float32),
            jax.ShapeDtypeStruct((NT, LSUB), jnp.float32),
        ],
        mesh=mesh,
        scratch_types=[
            pltpu.VMEM((CHUNK,), jnp.int32),
            pltpu.VMEM((CHUNK, D), jnp.float32),
            pltpu.VMEM((CHUNK, LSUB), jnp.float32),
            pltpu.SemaphoreType.DMA,
            pltpu.SemaphoreType.DMA,
        ],
    )
    def sc_scatter(x_hbm, pos_hbm, wrow_hbm, xs_hbm, ws_hbm,
                   idx_v, rows_v, wv, sem1, sem2):
        base = _sc_worker_base()
        pltpu.sync_copy(pos_hbm.at[pl.ds(base, CHUNK)], idx_v)
        pltpu.sync_copy(x_hbm.at[pl.ds(base, CHUNK)], rows_v)
        pltpu.sync_copy(wrow_hbm.at[pl.ds(base, CHUNK)], wv)
        cp1 = pltpu.async_copy(rows_v, xs_hbm.at[idx_v], sem1)
        cp2 = pltpu.async_copy(wv, ws_hbm.at[idx_v], sem2)
        cp1.wait()
        cp2.wait()

    @functools.partial(
        pl.kernel,
        out_type=jax.ShapeDtypeStruct((NT, D), jnp.float32),
        mesh=mesh,
        scratch_types=[
            pltpu.VMEM((CHUNK,), jnp.int32),
            pltpu.VMEM((CHUNK, D), jnp.float32),
            pltpu.SemaphoreType.DMA,
        ],
    )
    def sc_gather(y_hbm, pos_hbm, out_hbm, idx_v, rows_v, sem):
        base = _sc_worker_base()
        pltpu.sync_copy(pos_hbm.at[pl.ds(base, CHUNK)], idx_v)
        pltpu.async_copy(y_hbm.at[idx_v], rows_v, sem).wait()
        pltpu.sync_copy(rows_v, out_hbm.at[pl.ds(base, CHUNK)])

    return sc_scatter, sc_gather


def _ffn_body(meta_ref,
              xs_ref, w1_ref, b1_ref, w2_ref, b2_ref, ws_ref, out_ref):
    w = pl.program_id(0)
    rs = meta_ref[2, w]
    re = meta_ref[3, w]

    @pl.when(rs < re)
    def _work():
        xt = xs_ref[...].astype(jnp.bfloat16)
        h = jnp.dot(xt, w1_ref[0].astype(jnp.bfloat16),
                    preferred_element_type=jnp.float32)
        h = _gelu_exact(h + b1_ref[0])
        y = jnp.dot(h.astype(jnp.bfloat16), w2_ref[0].astype(jnp.bfloat16),
                    preferred_element_type=jnp.float32)
        y = (y + b2_ref[0]) * ws_ref[...][:, :1]
        row = lax.broadcasted_iota(jnp.int32, (BT, 1), 0)
        mask = (row >= rs) & (row < re)
        out_ref[...] = jnp.where(mask, y, out_ref[...])


def _grouped_ffn(meta, x_sorted, w1, b1, w2, b2, w_sorted):
    grid_spec = pltpu.PrefetchScalarGridSpec(
        num_scalar_prefetch=1,
        grid=(W_MAX,),
        in_specs=[
            pl.BlockSpec((BT, D), lambda w, m: (m[1, w], 0)),
            pl.BlockSpec((1, D, D), lambda w, m: (m[0, w], 0, 0)),
            pl.BlockSpec((1, 1, D), lambda w, m: (m[0, w], 0, 0)),
            pl.BlockSpec((1, D, D), lambda w, m: (m[0, w], 0, 0)),
            pl.BlockSpec((1, 1, D), lambda w, m: (m[0, w], 0, 0)),
            pl.BlockSpec((BT, LSUB), lambda w, m: (m[1, w], 0)),
        ],
        out_specs=pl.BlockSpec((BT, D), lambda w, m: (m[1, w], 0)),
    )
    return pl.pallas_call(
        _ffn_body,
        grid_spec=grid_spec,
        out_shape=jax.ShapeDtypeStruct((NT, D), jnp.float32),
        compiler_params=pltpu.CompilerParams(
            dimension_semantics=("arbitrary",),
        ),
    )(meta, x_sorted, w1, b1.reshape(E, 1, D), w2, b2.reshape(E, 1, D),
      w_sorted)


def kernel(x, gate_w, gate_b, w1, b1, w2, b2):
    n, t, d = x.shape
    x_flat = x.reshape(n * t, d)

    sc_scatter, sc_gather = _sc_kernels()
    pos2d, wrow, meta = _gating(x_flat, gate_w, gate_b)
    pos = pos2d.reshape(NT)
    x_sorted, w_sorted = sc_scatter(x_flat, pos, wrow)
    y_sorted = _grouped_ffn(meta, x_sorted, w1, b1, w2, b2, w_sorted)
    out = sc_gather(y_sorted, pos)
    return out.reshape(n, t, d)
